# Initial kernel scaffold; baseline (speedup 1.0000x reference)
import jax
import jax.numpy as jnp
from jax.experimental import pallas as pl


def kernel(numerical, node_feature, edge_index_dis, edge_index_od, node_mask, stage, params):
    raise NotImplementedError("write your pallas kernel here")


# speedup vs baseline: 12.0825x; 12.0825x over previous
import functools

import jax
import jax.numpy as jnp
import numpy as np
from jax import lax
from jax.experimental import pallas as pl
from jax.experimental.pallas import tpu as pltpu
from jax.experimental.pallas import tpu_sc as plsc

EPS = 1e-06
N_HEADS = 8
B = 8
N = 1177
E = 40000
D = 64
NP = 1280
N2 = 1184
W = 128
N_TILES = 16
CHUNK = 128
N_CHUNKS = 20
E_TILE = CHUNK * N_CHUNKS
E_PAD = E_TILE * N_TILES
NEG = -1e30


def _pre_body(nf_ref, pos_ref, nw_ref, nb_ref,
              a1_ref, a1b_ref, b1_ref, b1b_ref,
              a2_ref, a2b_ref, b2_ref, b2b_ref,
              zx_ref, hn_ref):
    nf = nf_ref[0]
    h = jnp.tanh(nf @ nw_ref[...] + nb_ref[...]) + pos_ref[0]
    hn_ref[0] = h
    rowmask = lax.broadcasted_iota(jnp.int32, (N2, W), 0) < N
    ones16 = jnp.ones((N2, 16), jnp.float32)
    lanepad = jnp.zeros((N2, W - D - 16), jnp.float32)
    z1 = jnp.tanh(jnp.tanh(h @ a1_ref[...] + a1b_ref[...]) @ b1_ref[...] + b1b_ref[...])
    z2 = jnp.tanh(jnp.tanh(h @ a2_ref[...] + a2b_ref[...]) @ b2_ref[...] + b2b_ref[...])
    pad = jnp.zeros((NP - N2, W), jnp.float32)
    zx_ref[0, 0, :N2, :] = jnp.where(
        rowmask, jnp.concatenate([z1, ones16, lanepad], axis=-1), 0.0)
    zx_ref[0, 0, N2:, :] = pad
    zx_ref[0, 1, :N2, :] = jnp.where(
        rowmask, jnp.concatenate([z2, ones16, lanepad], axis=-1), 0.0)
    zx_ref[0, 1, N2:, :] = pad


def _run_pre(nf_pad, pos_pad, node_enc, fc1, fc2):
    full = lambda shape: pl.BlockSpec(shape, lambda b: tuple(0 for _ in shape))
    return pl.pallas_call(
        _pre_body,
        grid=(B,),
        in_specs=[
            pl.BlockSpec((1, N2, 32), lambda b: (b, 0, 0)),
            pl.BlockSpec((1, N2, D), lambda b: (0, 0, 0)),
            full((32, D)), full((D,)),
            full((D, D)), full((D,)), full((D, D)), full((D,)),
            full((D, D)), full((D,)), full((D, D)), full((D,)),
        ],
        out_specs=[
            pl.BlockSpec((1, 2, NP, W), lambda b: (b, 0, 0, 0)),
            pl.BlockSpec((1, N2, D), lambda b: (b, 0, 0)),
        ],
        out_shape=[
            jax.ShapeDtypeStruct((B, 2, NP, W), jnp.float32),
            jax.ShapeDtypeStruct((B, N2, D), jnp.float32),
        ],
    )(nf_pad, pos_pad, node_enc['W'], node_enc['b'],
      fc1[0]['W'], fc1[0]['b'], fc1[1]['W'], fc1[1]['b'],
      fc2[0]['W'], fc2[0]['b'], fc2[1]['W'], fc2[1]['b'])


def _sc_body(zx_hbm, i0_hbm, i1_hbm, s0_hbm, s1_hbm, zz_hbm, out_hbm,
             g0_v, g1_v, l0_v, l1_v, r0_v, r1_v, acc_sh, sem0, sem1):
    c = lax.axis_index("c")
    t = lax.axis_index("s")
    for q in range(B):
        base = q * NP + t * 80
        pltpu.sync_copy(zz_hbm, acc_sh.at[pl.ds(base, 80)])
    plsc.subcore_barrier()
    for q in range(B):
        pltpu.sync_copy(i0_hbm.at[c, q, t], g0_v)
        pltpu.sync_copy(i1_hbm.at[c, q, t], g1_v)
        pltpu.sync_copy(s0_hbm.at[c, q, t], l0_v)
        pltpu.sync_copy(s1_hbm.at[c, q, t], l1_v)

        @pl.loop(0, N_CHUNKS)
        def _chunk(ch):
            cp0 = pltpu.async_copy(zx_hbm.at[g0_v.at[ch]], r0_v, sem0)
            cp1 = pltpu.async_copy(zx_hbm.at[g1_v.at[ch]], r1_v, sem1)
            cp0.wait()
            cp1.wait()
            pltpu.sync_copy(r1_v, acc_sh.at[l0_v.at[ch]], add=True)
            pltpu.sync_copy(r0_v, acc_sh.at[l1_v.at[ch]], add=True)

    plsc.subcore_barrier()
    for q in range(B):
        base = q * NP + t * 80
        pltpu.sync_copy(acc_sh.at[pl.ds(base, 80)],
                        out_hbm.at[q, c, pl.ds(t * 80, 80)])


@functools.lru_cache(maxsize=1)
def _get_sc_call():
    return functools.partial(
        pl.kernel,
        out_type=jax.ShapeDtypeStruct((B, 2, NP, W), jnp.float32),
        mesh=plsc.VectorSubcoreMesh(core_axis_name="c", subcore_axis_name="s"),
        scratch_types=[
            pltpu.VMEM((N_CHUNKS, CHUNK), jnp.int32),
            pltpu.VMEM((N_CHUNKS, CHUNK), jnp.int32),
            pltpu.VMEM((N_CHUNKS, CHUNK), jnp.int32),
            pltpu.VMEM((N_CHUNKS, CHUNK), jnp.int32),
            pltpu.VMEM((CHUNK, W), jnp.float32),
            pltpu.VMEM((CHUNK, W), jnp.float32),
            pltpu.VMEM_SHARED((B * NP, W), jnp.float32),
            pltpu.SemaphoreType.DMA,
            pltpu.SemaphoreType.DMA,
        ],
    )(_sc_body)


def _prep_edges(edge_index_dis, edge_index_od):
    def one(ei, s):
        pad = jnp.full((B, E_PAD - E, 2), N, jnp.int32)
        eip = jnp.concatenate([ei.astype(jnp.int32), pad], axis=1)
        gbase = ((2 * jnp.arange(B, dtype=jnp.int32) + s) * NP)[:, None]
        lbase = (jnp.arange(B, dtype=jnp.int32) * NP)[:, None]
        shp = (B, N_TILES, N_CHUNKS, CHUNK)
        return ((eip[:, :, 0] + gbase).reshape(shp),
                (eip[:, :, 1] + gbase).reshape(shp),
                (eip[:, :, 0] + lbase).reshape(shp),
                (eip[:, :, 1] + lbase).reshape(shp))
    d = one(edge_index_dis, 0)
    o = one(edge_index_od, 1)
    return tuple(jnp.stack([d[k], o[k]], axis=0) for k in range(4))


def _post_body(sc_ref, hn_ref, num_ref, stage_ref, mc_ref,
               gw_ref, gb_ref, uw_ref, ub_ref,
               qw_ref, qb_ref, kw_ref, kb_ref, vw_ref, vb_ref,
               mqw_ref, mqb_ref, mkw_ref, mkb_ref, mvw_ref, mvb_ref,
               mow_ref, mob_ref, lg_ref, lb_ref,
               n1w_ref, n1b_ref, n2w_ref, n2b_ref,
               hatt_ref, sv_ref):
    hnum = jnp.tanh(num_ref[0] @ n1w_ref[...] + n1b_ref[...])
    hnum = jnp.tanh(hnum @ n2w_ref[...] + n2b_ref[...])
    cur = hn_ref[0]

    hs = []
    for s in range(2):
        acc = sc_ref[0, s]
        comb = acc[:, :D] / (acc[:, D:D + 1] + EPS)
        cat = jnp.concatenate([cur, comb], axis=-1)
        gate = jnp.tanh(cat @ gw_ref[...] + gb_ref[...])
        upd = jnp.tanh(cat @ uw_ref[...] + ub_ref[...])
        hs.append(gate * upd + (1.0 - gate) * cur)
    hcat = jnp.concatenate(hs, axis=-1)

    q = hcat @ qw_ref[...] + qb_ref[...]
    k = hcat @ kw_ref[...] + kb_ref[...]
    v = hcat @ vw_ref[...] + vb_ref[...]
    qp = q @ mqw_ref[...] + mqb_ref[...]
    kp = k @ mkw_ref[...] + mkb_ref[...]
    vp = v @ mvw_ref[...] + mvb_ref[...]

    keymask = (mc_ref[...] > 0.0)
    dh = D // N_HEADS
    scale = 1.0 / float(np.sqrt(dh))
    outs = []
    for h in range(N_HEADS):
        sl = slice(h * dh, (h + 1) * dh)
        s_ = lax.dot_general(qp[:, sl], kp[:, sl],
                             (((1,), (1,)), ((), ()))) * scale
        s_ = jnp.where(keymask, s_, NEG)
        m = jnp.max(s_, axis=-1, keepdims=True)
        e = jnp.exp(s_ - m)
        a = e / jnp.sum(e, axis=-1, keepdims=True)
        outs.append(a @ vp[:, sl])
    att = jnp.concatenate(outs, axis=-1) @ mow_ref[...] + mob_ref[...]

    mu = att.mean(axis=-1, keepdims=True)
    var = ((att - mu) ** 2).mean(axis=-1, keepdims=True)
    h_att = (att - mu) / jnp.sqrt(var + 1e-05) * lg_ref[...] + lb_ref[...]
    hatt_ref[0] = h_att

    rmask = lax.broadcasted_iota(jnp.int32, (N2, D), 0) < N
    hmean = jnp.sum(jnp.where(rmask, h_att, 0.0), axis=0, keepdims=True) / float(N)
    sv_ref[0, 0, :D] = hnum[0]
    sv_ref[0, 0, D:2 * D] = hmean[0]
    sv_ref[0, 0, 2 * D:] = stage_ref[0, 0]


def _run_post(sc_out, hn, numerical, stage, maskcol, params):
    full = lambda shape: pl.BlockSpec(shape, lambda b: tuple(0 for _ in shape))
    g = params['gated']
    m = params['mha1']
    ne = params['num_enc']
    return pl.pallas_call(
        _post_body,
        grid=(B,),
        in_specs=[
            pl.BlockSpec((1, 2, N2, W), lambda b: (b, 0, 0, 0)),
            pl.BlockSpec((1, N2, D), lambda b: (b, 0, 0)),
            pl.BlockSpec((1, 1, 64), lambda b: (b, 0, 0)),
            pl.BlockSpec((1, 1, 2), lambda b: (b, 0, 0)),
            full((1, N2)),
            full((2 * D, D)), full((D,)), full((2 * D, D)), full((D,)),
            full((2 * D, D)), full((D,)), full((2 * D, D)), full((D,)),
            full((2 * D, D)), full((D,)),
            full((D, D)), full((D,)), full((D, D)), full((D,)),
            full((D, D)), full((D,)), full((D, D)), full((D,)),
            full((D,)), full((D,)),
            full((64, 128)), full((128,)), full((128, 64)), full((64,)),
        ],
        out_specs=[
            pl.BlockSpec((1, N2, D), lambda b: (b, 0, 0)),
            pl.BlockSpec((1, 1, 2 * D + 2), lambda b: (b, 0, 0)),
        ],
        out_shape=[
            jax.ShapeDtypeStruct((B, N2, D), jnp.float32),
            jax.ShapeDtypeStruct((B, 1, 2 * D + 2), jnp.float32),
        ],
    )(sc_out, hn, numerical.reshape(B, 1, -1), stage.reshape(B, 1, 2), maskcol,
      g['gate_W'], g['gate_b'], g['update_W'], g['update_b'],
      params['attn_q']['W'], params['attn_q']['b'],
      params['attn_k']['W'], params['attn_k']['b'],
      params['attn_v']['W'], params['attn_v']['b'],
      m['Wq'], m['bq'], m['Wk'], m['bk'], m['Wv'], m['bv'], m['Wo'], m['bo'],
      params['ln1']['g'], params['ln1']['b'],
      ne[0]['W'], ne[0]['b'], ne[1]['W'], ne[1]['b'])


def kernel(numerical, node_feature, edge_index_dis, edge_index_od, node_mask, stage, params):
    nf_pad = jnp.pad(node_feature, ((0, 0), (0, N2 - N), (0, 0)))
    pos_pad = jnp.pad(params['pos_enc'][:, :N, :], ((0, 0), (0, N2 - N), (0, 0)))
    zx, hn = _run_pre(nf_pad, pos_pad, params['node_enc'],
                      params['edge_fc1'][-1], params['edge_fc2'][-1])
    i0, i1, s0, s1 = _prep_edges(edge_index_dis, edge_index_od)
    zeros_in = jnp.zeros((80, W), jnp.float32)
    sc_out = _get_sc_call()(zx.reshape(2 * B * NP, W), i0, i1, s0, s1, zeros_in)
    maskcol = nf_pad[0:1, :, -5]
    h_att, sv = _run_post(sc_out[:, :, :N2, :], hn,
                          numerical.reshape(B, -1), stage, maskcol, params)
    return (h_att[:, :N, :], sv[:, 0, :], node_mask, stage)


# speedup vs baseline: 13.1312x; 1.0868x over previous
import functools

import jax
import jax.numpy as jnp
import numpy as np
from jax import lax
from jax.experimental import pallas as pl
from jax.experimental.pallas import tpu as pltpu
from jax.experimental.pallas import tpu_sc as plsc

EPS = 1e-06
N_HEADS = 8
B = 8
N = 1177
E = 40000
D = 64
NP = 1280
N2 = 1184
W = 128
N_TILES = 16
CHUNK = 128
N_CHUNKS = 20
E_TILE = CHUNK * N_CHUNKS
E_PAD = E_TILE * N_TILES
WAVE = 4
NEG = -1e30


def _pre_body(nf_ref, pos_ref, nw_ref, nb_ref,
              a1_ref, a1b_ref, b1_ref, b1b_ref,
              a2_ref, a2b_ref, b2_ref, b2b_ref,
              zx_ref, hn_ref):
    nf = nf_ref[0]
    h = jnp.tanh(nf @ nw_ref[...] + nb_ref[...]) + pos_ref[0]
    hn_ref[0] = h
    rowmask = lax.broadcasted_iota(jnp.int32, (N2, W), 0) < N
    ones16 = jnp.ones((N2, 16), jnp.float32)
    lanepad = jnp.zeros((N2, W - D - 16), jnp.float32)
    z1 = jnp.tanh(jnp.tanh(h @ a1_ref[...] + a1b_ref[...]) @ b1_ref[...] + b1b_ref[...])
    z2 = jnp.tanh(jnp.tanh(h @ a2_ref[...] + a2b_ref[...]) @ b2_ref[...] + b2b_ref[...])
    pad = jnp.zeros((NP - N2, W), jnp.float32)
    zx_ref[0, 0, :N2, :] = jnp.where(
        rowmask, jnp.concatenate([z1, ones16, lanepad], axis=-1), 0.0)
    zx_ref[0, 0, N2:, :] = pad
    zx_ref[0, 1, :N2, :] = jnp.where(
        rowmask, jnp.concatenate([z2, ones16, lanepad], axis=-1), 0.0)
    zx_ref[0, 1, N2:, :] = pad


def _run_pre(nf_pad, pos_pad, node_enc, fc1, fc2):
    full = lambda shape: pl.BlockSpec(shape, lambda b: tuple(0 for _ in shape))
    return pl.pallas_call(
        _pre_body,
        grid=(B,),
        in_specs=[
            pl.BlockSpec((1, N2, 32), lambda b: (b, 0, 0)),
            pl.BlockSpec((1, N2, D), lambda b: (0, 0, 0)),
            full((32, D)), full((D,)),
            full((D, D)), full((D,)), full((D, D)), full((D,)),
            full((D, D)), full((D,)), full((D, D)), full((D,)),
        ],
        out_specs=[
            pl.BlockSpec((1, 2, NP, W), lambda b: (b, 0, 0, 0)),
            pl.BlockSpec((1, N2, D), lambda b: (b, 0, 0)),
        ],
        out_shape=[
            jax.ShapeDtypeStruct((B, 2, NP, W), jnp.float32),
            jax.ShapeDtypeStruct((B, N2, D), jnp.float32),
        ],
    )(nf_pad, pos_pad, node_enc['W'], node_enc['b'],
      fc1[0]['W'], fc1[0]['b'], fc1[1]['W'], fc1[1]['b'],
      fc2[0]['W'], fc2[0]['b'], fc2[1]['W'], fc2[1]['b'])


def _sc_body(zx_hbm, i0_hbm, i1_hbm, s0_hbm, s1_hbm, zz_hbm, out_hbm,
             g0_v, g1_v, l0_v, l1_v, r0x, r1x, r0y, r1y, acc_sh,
             sgx0, sgx1, sgy0, sgy1, ssx0, ssx1, ssy0, ssy1):
    c = lax.axis_index("c")
    t = lax.axis_index("s")

    def wait_gather(idx_row, dst, sem):
        pltpu.make_async_copy(zx_hbm.at[idx_row], dst, sem).wait()

    def wait_scatter(dst, sem):
        pltpu.make_async_copy(zx_hbm.at[pl.ds(0, CHUNK)], dst, sem).wait()

    def do_pair(q):
        pltpu.sync_copy(i0_hbm.at[c, q, t], g0_v)
        pltpu.sync_copy(i1_hbm.at[c, q, t], g1_v)
        pltpu.sync_copy(s0_hbm.at[c, q, t], l0_v)
        pltpu.sync_copy(s1_hbm.at[c, q, t], l1_v)

        pltpu.async_copy(zx_hbm.at[g0_v.at[0]], r0x, sgx0)
        pltpu.async_copy(zx_hbm.at[g1_v.at[0]], r1x, sgx1)

        @pl.loop(0, N_CHUNKS // 2)
        def _k(k):
            ch0 = 2 * k
            ch1 = 2 * k + 1
            wait_gather(g0_v.at[ch0], r0x, sgx0)
            wait_gather(g1_v.at[ch0], r1x, sgx1)
            pltpu.async_copy(r1x, acc_sh.at[l0_v.at[ch0]], ssx0, add=True)
            pltpu.async_copy(r0x, acc_sh.at[l1_v.at[ch0]], ssx1, add=True)

            @pl.when(k > 0)
            def _():
                wait_scatter(r0y, ssy0)
                wait_scatter(r1y, ssy1)

            pltpu.async_copy(zx_hbm.at[g0_v.at[ch1]], r0y, sgy0)
            pltpu.async_copy(zx_hbm.at[g1_v.at[ch1]], r1y, sgy1)
            wait_gather(g0_v.at[ch1], r0y, sgy0)
            wait_gather(g1_v.at[ch1], r1y, sgy1)
            pltpu.async_copy(r1y, acc_sh.at[l0_v.at[ch1]], ssy0, add=True)
            pltpu.async_copy(r0y, acc_sh.at[l1_v.at[ch1]], ssy1, add=True)
            wait_scatter(r0x, ssx0)
            wait_scatter(r1x, ssx1)

            @pl.when(k < N_CHUNKS // 2 - 1)
            def _():
                pltpu.async_copy(zx_hbm.at[g0_v.at[ch0 + 2]], r0x, sgx0)
                pltpu.async_copy(zx_hbm.at[g1_v.at[ch0 + 2]], r1x, sgx1)

        wait_scatter(r0y, ssy0)
        wait_scatter(r1y, ssy1)

    for wave in range(2):
        for q in range(WAVE):
            base = q * NP + t * 80
            pltpu.sync_copy(zz_hbm, acc_sh.at[pl.ds(base, 80)])
        plsc.subcore_barrier()
        for q in range(WAVE):
            do_pair(wave * WAVE + q)
        plsc.subcore_barrier()
        for q in range(WAVE):
            base = q * NP + t * 80
            pltpu.sync_copy(acc_sh.at[pl.ds(base, 80)],
                            out_hbm.at[wave * WAVE + q, c, pl.ds(t * 80, 80)])


@functools.lru_cache(maxsize=1)
def _get_sc_call():
    return functools.partial(
        pl.kernel,
        out_type=jax.ShapeDtypeStruct((B, 2, NP, W), jnp.float32),
        mesh=plsc.VectorSubcoreMesh(core_axis_name="c", subcore_axis_name="s"),
        scratch_types=[
            pltpu.VMEM((N_CHUNKS, CHUNK), jnp.int32),
            pltpu.VMEM((N_CHUNKS, CHUNK), jnp.int32),
            pltpu.VMEM((N_CHUNKS, CHUNK), jnp.int32),
            pltpu.VMEM((N_CHUNKS, CHUNK), jnp.int32),
            pltpu.VMEM((CHUNK, W), jnp.float32),
            pltpu.VMEM((CHUNK, W), jnp.float32),
            pltpu.VMEM((CHUNK, W), jnp.float32),
            pltpu.VMEM((CHUNK, W), jnp.float32),
            pltpu.VMEM_SHARED((WAVE * NP, W), jnp.float32),
            pltpu.SemaphoreType.DMA,
            pltpu.SemaphoreType.DMA,
            pltpu.SemaphoreType.DMA,
            pltpu.SemaphoreType.DMA,
            pltpu.SemaphoreType.DMA,
            pltpu.SemaphoreType.DMA,
            pltpu.SemaphoreType.DMA,
            pltpu.SemaphoreType.DMA,
        ],
    )(_sc_body)


def _prep_edges(edge_index_dis, edge_index_od):
    def one(ei, s):
        pad = jnp.full((B, E_PAD - E, 2), N, jnp.int32)
        eip = jnp.concatenate([ei.astype(jnp.int32), pad], axis=1)
        gbase = ((2 * jnp.arange(B, dtype=jnp.int32) + s) * NP)[:, None]
        lbase = ((jnp.arange(B, dtype=jnp.int32) % WAVE) * NP)[:, None]
        shp = (B, N_TILES, N_CHUNKS, CHUNK)
        return ((eip[:, :, 0] + gbase).reshape(shp),
                (eip[:, :, 1] + gbase).reshape(shp),
                (eip[:, :, 0] + lbase).reshape(shp),
                (eip[:, :, 1] + lbase).reshape(shp))
    d = one(edge_index_dis, 0)
    o = one(edge_index_od, 1)
    return tuple(jnp.stack([d[k], o[k]], axis=0) for k in range(4))


def _post_body(sc_ref, hn_ref, num_ref, stage_ref, mc_ref,
               gw_ref, gb_ref, uw_ref, ub_ref,
               qw_ref, qb_ref, kw_ref, kb_ref, vw_ref, vb_ref,
               mqw_ref, mqb_ref, mkw_ref, mkb_ref, mvw_ref, mvb_ref,
               mow_ref, mob_ref, lg_ref, lb_ref,
               n1w_ref, n1b_ref, n2w_ref, n2b_ref,
               hatt_ref, sv_ref):
    hnum = jnp.tanh(num_ref[0] @ n1w_ref[...] + n1b_ref[...])
    hnum = jnp.tanh(hnum @ n2w_ref[...] + n2b_ref[...])
    cur = hn_ref[0]

    hs = []
    for s in range(2):
        acc = sc_ref[0, s]
        comb = acc[:, :D] / (acc[:, D:D + 1] + EPS)
        cat = jnp.concatenate([cur, comb], axis=-1)
        gate = jnp.tanh(cat @ gw_ref[...] + gb_ref[...])
        upd = jnp.tanh(cat @ uw_ref[...] + ub_ref[...])
        hs.append(gate * upd + (1.0 - gate) * cur)
    hcat = jnp.concatenate(hs, axis=-1)

    q = hcat @ qw_ref[...] + qb_ref[...]
    k = hcat @ kw_ref[...] + kb_ref[...]
    v = hcat @ vw_ref[...] + vb_ref[...]
    qp = q @ mqw_ref[...] + mqb_ref[...]
    kp = k @ mkw_ref[...] + mkb_ref[...]
    vp = v @ mvw_ref[...] + mvb_ref[...]

    keymask = (mc_ref[...] > 0.0)
    dh = D // N_HEADS
    scale = 1.0 / float(np.sqrt(dh))
    outs = []
    for h in range(N_HEADS):
        sl = slice(h * dh, (h + 1) * dh)
        s_ = lax.dot_general(qp[:, sl], kp[:, sl],
                             (((1,), (1,)), ((), ()))) * scale
        s_ = jnp.where(keymask, s_, NEG)
        m = jnp.max(s_, axis=-1, keepdims=True)
        e = jnp.exp(s_ - m)
        a = e / jnp.sum(e, axis=-1, keepdims=True)
        outs.append(a @ vp[:, sl])
    att = jnp.concatenate(outs, axis=-1) @ mow_ref[...] + mob_ref[...]

    mu = att.mean(axis=-1, keepdims=True)
    var = ((att - mu) ** 2).mean(axis=-1, keepdims=True)
    h_att = (att - mu) / jnp.sqrt(var + 1e-05) * lg_ref[...] + lb_ref[...]
    hatt_ref[0] = h_att

    rmask = lax.broadcasted_iota(jnp.int32, (N2, D), 0) < N
    hmean = jnp.sum(jnp.where(rmask, h_att, 0.0), axis=0, keepdims=True) / float(N)
    sv_ref[0, 0, :D] = hnum[0]
    sv_ref[0, 0, D:2 * D] = hmean[0]
    sv_ref[0, 0, 2 * D:] = stage_ref[0, 0]


def _run_post(sc_out, hn, numerical, stage, maskcol, params):
    full = lambda shape: pl.BlockSpec(shape, lambda b: tuple(0 for _ in shape))
    g = params['gated']
    m = params['mha1']
    ne = params['num_enc']
    return pl.pallas_call(
        _post_body,
        grid=(B,),
        in_specs=[
            pl.BlockSpec((1, 2, N2, W), lambda b: (b, 0, 0, 0)),
            pl.BlockSpec((1, N2, D), lambda b: (b, 0, 0)),
            pl.BlockSpec((1, 1, 64), lambda b: (b, 0, 0)),
            pl.BlockSpec((1, 1, 2), lambda b: (b, 0, 0)),
            full((1, N2)),
            full((2 * D, D)), full((D,)), full((2 * D, D)), full((D,)),
            full((2 * D, D)), full((D,)), full((2 * D, D)), full((D,)),
            full((2 * D, D)), full((D,)),
            full((D, D)), full((D,)), full((D, D)), full((D,)),
            full((D, D)), full((D,)), full((D, D)), full((D,)),
            full((D,)), full((D,)),
            full((64, 128)), full((128,)), full((128, 64)), full((64,)),
        ],
        out_specs=[
            pl.BlockSpec((1, N2, D), lambda b: (b, 0, 0)),
            pl.BlockSpec((1, 1, 2 * D + 2), lambda b: (b, 0, 0)),
        ],
        out_shape=[
            jax.ShapeDtypeStruct((B, N2, D), jnp.float32),
            jax.ShapeDtypeStruct((B, 1, 2 * D + 2), jnp.float32),
        ],
    )(sc_out, hn, numerical.reshape(B, 1, -1), stage.reshape(B, 1, 2), maskcol,
      g['gate_W'], g['gate_b'], g['update_W'], g['update_b'],
      params['attn_q']['W'], params['attn_q']['b'],
      params['attn_k']['W'], params['attn_k']['b'],
      params['attn_v']['W'], params['attn_v']['b'],
      m['Wq'], m['bq'], m['Wk'], m['bk'], m['Wv'], m['bv'], m['Wo'], m['bo'],
      params['ln1']['g'], params['ln1']['b'],
      ne[0]['W'], ne[0]['b'], ne[1]['W'], ne[1]['b'])


def kernel(numerical, node_feature, edge_index_dis, edge_index_od, node_mask, stage, params):
    nf_pad = jnp.pad(node_feature, ((0, 0), (0, N2 - N), (0, 0)))
    pos_pad = jnp.pad(params['pos_enc'][:, :N, :], ((0, 0), (0, N2 - N), (0, 0)))
    zx, hn = _run_pre(nf_pad, pos_pad, params['node_enc'],
                      params['edge_fc1'][-1], params['edge_fc2'][-1])
    i0, i1, s0, s1 = _prep_edges(edge_index_dis, edge_index_od)
    zeros_in = jnp.zeros((80, W), jnp.float32)
    sc_out = _get_sc_call()(zx.reshape(2 * B * NP, W), i0, i1, s0, s1, zeros_in)
    maskcol = nf_pad[0:1, :, -5]
    h_att, sv = _run_post(sc_out[:, :, :N2, :], hn,
                          numerical.reshape(B, -1), stage, maskcol, params)
    return (h_att[:, :N, :], sv[:, 0, :], node_mask, stage)


# speedup vs baseline: 18.9275x; 1.4414x over previous
import functools

import jax
import jax.numpy as jnp
import numpy as np
from jax import lax
from jax.experimental import pallas as pl
from jax.experimental.pallas import tpu as pltpu
from jax.experimental.pallas import tpu_sc as plsc

EPS = 1e-06
N_HEADS = 8
B = 8
N = 1177
E = 40000
D = 64
NP = 1280
N2 = 1184
W = 80
N_TILES = 16
CHUNK = 128
N_CHUNKS = 20
E_TILE = CHUNK * N_CHUNKS
E_PAD = E_TILE * N_TILES
WAVE = 4
NEG = -1e30


def _pre_body(nf_ref, pos_ref, nw_ref, nb_ref,
              a1_ref, a1b_ref, b1_ref, b1b_ref,
              a2_ref, a2b_ref, b2_ref, b2b_ref,
              zx_ref, hn_ref):
    nf = nf_ref[0]
    h = jnp.tanh(nf @ nw_ref[...] + nb_ref[...]) + pos_ref[0]
    hn_ref[0] = h
    rowmask = lax.broadcasted_iota(jnp.int32, (N2, W), 0) < N
    widen = [jnp.ones((N2, 16), jnp.float32)]
    if W > D + 16:
        widen.append(jnp.zeros((N2, W - D - 16), jnp.float32))
    z1 = jnp.tanh(jnp.tanh(h @ a1_ref[...] + a1b_ref[...]) @ b1_ref[...] + b1b_ref[...])
    z2 = jnp.tanh(jnp.tanh(h @ a2_ref[...] + a2b_ref[...]) @ b2_ref[...] + b2b_ref[...])
    pad = jnp.zeros((NP - N2, W), jnp.float32)
    zx_ref[0, 0, :N2, :] = jnp.where(
        rowmask, jnp.concatenate([z1] + widen, axis=-1), 0.0)
    zx_ref[0, 0, N2:, :] = pad
    zx_ref[0, 1, :N2, :] = jnp.where(
        rowmask, jnp.concatenate([z2] + widen, axis=-1), 0.0)
    zx_ref[0, 1, N2:, :] = pad


def _run_pre(nf_pad, pos_pad, node_enc, fc1, fc2):
    full = lambda shape: pl.BlockSpec(shape, lambda b: tuple(0 for _ in shape))
    return pl.pallas_call(
        _pre_body,
        grid=(B,),
        in_specs=[
            pl.BlockSpec((1, N2, 32), lambda b: (b, 0, 0)),
            pl.BlockSpec((1, N2, D), lambda b: (0, 0, 0)),
            full((32, D)), full((D,)),
            full((D, D)), full((D,)), full((D, D)), full((D,)),
            full((D, D)), full((D,)), full((D, D)), full((D,)),
        ],
        out_specs=[
            pl.BlockSpec((1, 2, NP, W), lambda b: (b, 0, 0, 0)),
            pl.BlockSpec((1, N2, D), lambda b: (b, 0, 0)),
        ],
        out_shape=[
            jax.ShapeDtypeStruct((B, 2, NP, W), jnp.float32),
            jax.ShapeDtypeStruct((B, N2, D), jnp.float32),
        ],
    )(nf_pad, pos_pad, node_enc['W'], node_enc['b'],
      fc1[0]['W'], fc1[0]['b'], fc1[1]['W'], fc1[1]['b'],
      fc2[0]['W'], fc2[0]['b'], fc2[1]['W'], fc2[1]['b'])


def _sc_body(zx_hbm, i0_hbm, i1_hbm, s0_hbm, s1_hbm, zz_hbm, out_hbm,
             g0_v, g1_v, l0_v, l1_v, r0x, r1x, r0y, r1y, acc_sh,
             sgx0, sgx1, sgy0, sgy1, ssx0, ssx1, ssy0, ssy1):
    c = lax.axis_index("c")
    t = lax.axis_index("s")

    def wait_gather(idx_row, dst, sem):
        pltpu.make_async_copy(zx_hbm.at[idx_row], dst, sem).wait()

    def wait_scatter(dst, sem):
        pltpu.make_async_copy(zx_hbm.at[pl.ds(0, CHUNK)], dst, sem).wait()

    def do_pair(q):
        pltpu.sync_copy(i0_hbm.at[c, q, t], g0_v)
        pltpu.sync_copy(i1_hbm.at[c, q, t], g1_v)
        pltpu.sync_copy(s0_hbm.at[c, q, t], l0_v)
        pltpu.sync_copy(s1_hbm.at[c, q, t], l1_v)

        pltpu.async_copy(zx_hbm.at[g0_v.at[0]], r0x, sgx0)
        pltpu.async_copy(zx_hbm.at[g1_v.at[0]], r1x, sgx1)

        @pl.loop(0, N_CHUNKS // 2)
        def _k(k):
            ch0 = 2 * k
            ch1 = 2 * k + 1
            wait_gather(g0_v.at[ch0], r0x, sgx0)
            wait_gather(g1_v.at[ch0], r1x, sgx1)
            pltpu.async_copy(r1x, acc_sh.at[l0_v.at[ch0]], ssx0, add=True)
            pltpu.async_copy(r0x, acc_sh.at[l1_v.at[ch0]], ssx1, add=True)

            @pl.when(k > 0)
            def _():
                wait_scatter(r0y, ssy0)
                wait_scatter(r1y, ssy1)

            pltpu.async_copy(zx_hbm.at[g0_v.at[ch1]], r0y, sgy0)
            pltpu.async_copy(zx_hbm.at[g1_v.at[ch1]], r1y, sgy1)
            wait_gather(g0_v.at[ch1], r0y, sgy0)
            wait_gather(g1_v.at[ch1], r1y, sgy1)
            pltpu.async_copy(r1y, acc_sh.at[l0_v.at[ch1]], ssy0, add=True)
            pltpu.async_copy(r0y, acc_sh.at[l1_v.at[ch1]], ssy1, add=True)
            wait_scatter(r0x, ssx0)
            wait_scatter(r1x, ssx1)

            @pl.when(k < N_CHUNKS // 2 - 1)
            def _():
                pltpu.async_copy(zx_hbm.at[g0_v.at[ch0 + 2]], r0x, sgx0)
                pltpu.async_copy(zx_hbm.at[g1_v.at[ch0 + 2]], r1x, sgx1)

        wait_scatter(r0y, ssy0)
        wait_scatter(r1y, ssy1)

    for wave in range(2):
        for q in range(WAVE):
            base = q * NP + t * 80
            pltpu.sync_copy(zz_hbm, acc_sh.at[pl.ds(base, 80)])
        plsc.subcore_barrier()
        for q in range(WAVE):
            do_pair(wave * WAVE + q)
        plsc.subcore_barrier()
        for q in range(WAVE):
            base = q * NP + t * 80
            pltpu.sync_copy(acc_sh.at[pl.ds(base, 80)],
                            out_hbm.at[wave * WAVE + q, c, pl.ds(t * 80, 80)])


@functools.lru_cache(maxsize=1)
def _get_sc_call():
    return functools.partial(
        pl.kernel,
        out_type=jax.ShapeDtypeStruct((B, 2, NP, W), jnp.float32),
        mesh=plsc.VectorSubcoreMesh(core_axis_name="c", subcore_axis_name="s"),
        compiler_params=pltpu.CompilerParams(use_tc_tiling_on_sc=False),
        scratch_types=[
            pltpu.VMEM((N_CHUNKS, CHUNK), jnp.int32),
            pltpu.VMEM((N_CHUNKS, CHUNK), jnp.int32),
            pltpu.VMEM((N_CHUNKS, CHUNK), jnp.int32),
            pltpu.VMEM((N_CHUNKS, CHUNK), jnp.int32),
            pltpu.VMEM((CHUNK, W), jnp.float32),
            pltpu.VMEM((CHUNK, W), jnp.float32),
            pltpu.VMEM((CHUNK, W), jnp.float32),
            pltpu.VMEM((CHUNK, W), jnp.float32),
            pltpu.VMEM_SHARED((WAVE * NP, W), jnp.float32),
            pltpu.SemaphoreType.DMA,
            pltpu.SemaphoreType.DMA,
            pltpu.SemaphoreType.DMA,
            pltpu.SemaphoreType.DMA,
            pltpu.SemaphoreType.DMA,
            pltpu.SemaphoreType.DMA,
            pltpu.SemaphoreType.DMA,
            pltpu.SemaphoreType.DMA,
        ],
    )(_sc_body)


def _prep_edges(edge_index_dis, edge_index_od):
    def one(ei, s):
        pad = jnp.full((B, E_PAD - E, 2), N, jnp.int32)
        eip = jnp.concatenate([ei.astype(jnp.int32), pad], axis=1)
        gbase = ((2 * jnp.arange(B, dtype=jnp.int32) + s) * NP)[:, None]
        lbase = ((jnp.arange(B, dtype=jnp.int32) % WAVE) * NP)[:, None]
        shp = (B, N_TILES, N_CHUNKS, CHUNK)
        return ((eip[:, :, 0] + gbase).reshape(shp),
                (eip[:, :, 1] + gbase).reshape(shp),
                (eip[:, :, 0] + lbase).reshape(shp),
                (eip[:, :, 1] + lbase).reshape(shp))
    d = one(edge_index_dis, 0)
    o = one(edge_index_od, 1)
    return tuple(jnp.stack([d[k], o[k]], axis=0) for k in range(4))


def _post_body(sc_ref, hn_ref, num_ref, stage_ref, mc_ref,
               gw_ref, gb_ref, uw_ref, ub_ref,
               qw_ref, qb_ref, kw_ref, kb_ref, vw_ref, vb_ref,
               mqw_ref, mqb_ref, mkw_ref, mkb_ref, mvw_ref, mvb_ref,
               mow_ref, mob_ref, lg_ref, lb_ref,
               n1w_ref, n1b_ref, n2w_ref, n2b_ref,
               hatt_ref, sv_ref):
    hnum = jnp.tanh(num_ref[0] @ n1w_ref[...] + n1b_ref[...])
    hnum = jnp.tanh(hnum @ n2w_ref[...] + n2b_ref[...])
    cur = hn_ref[0]

    hs = []
    for s in range(2):
        acc = sc_ref[0, s]
        comb = acc[:, :D] / (acc[:, D:D + 1] + EPS)
        cat = jnp.concatenate([cur, comb], axis=-1)
        gate = jnp.tanh(cat @ gw_ref[...] + gb_ref[...])
        upd = jnp.tanh(cat @ uw_ref[...] + ub_ref[...])
        hs.append(gate * upd + (1.0 - gate) * cur)
    hcat = jnp.concatenate(hs, axis=-1)

    q = hcat @ qw_ref[...] + qb_ref[...]
    k = hcat @ kw_ref[...] + kb_ref[...]
    v = hcat @ vw_ref[...] + vb_ref[...]
    qp = q @ mqw_ref[...] + mqb_ref[...]
    kp = k @ mkw_ref[...] + mkb_ref[...]
    vp = v @ mvw_ref[...] + mvb_ref[...]

    keymask = (mc_ref[...] > 0.0)
    dh = D // N_HEADS
    scale = 1.0 / float(np.sqrt(dh))
    outs = []
    for h in range(N_HEADS):
        sl = slice(h * dh, (h + 1) * dh)
        s_ = lax.dot_general(qp[:, sl], kp[:, sl],
                             (((1,), (1,)), ((), ()))) * scale
        s_ = jnp.where(keymask, s_, NEG)
        m = jnp.max(s_, axis=-1, keepdims=True)
        e = jnp.exp(s_ - m)
        a = e / jnp.sum(e, axis=-1, keepdims=True)
        outs.append(a @ vp[:, sl])
    att = jnp.concatenate(outs, axis=-1) @ mow_ref[...] + mob_ref[...]

    mu = att.mean(axis=-1, keepdims=True)
    var = ((att - mu) ** 2).mean(axis=-1, keepdims=True)
    h_att = (att - mu) / jnp.sqrt(var + 1e-05) * lg_ref[...] + lb_ref[...]
    hatt_ref[0] = h_att

    rmask = lax.broadcasted_iota(jnp.int32, (N2, D), 0) < N
    hmean = jnp.sum(jnp.where(rmask, h_att, 0.0), axis=0, keepdims=True) / float(N)
    sv_ref[0, 0, :D] = hnum[0]
    sv_ref[0, 0, D:2 * D] = hmean[0]
    sv_ref[0, 0, 2 * D:] = stage_ref[0, 0]


def _run_post(sc_out, hn, numerical, stage, maskcol, params):
    full = lambda shape: pl.BlockSpec(shape, lambda b: tuple(0 for _ in shape))
    g = params['gated']
    m = params['mha1']
    ne = params['num_enc']
    return pl.pallas_call(
        _post_body,
        grid=(B,),
        in_specs=[
            pl.BlockSpec((1, 2, N2, W), lambda b: (b, 0, 0, 0)),
            pl.BlockSpec((1, N2, D), lambda b: (b, 0, 0)),
            pl.BlockSpec((1, 1, 64), lambda b: (b, 0, 0)),
            pl.BlockSpec((1, 1, 2), lambda b: (b, 0, 0)),
            full((1, N2)),
            full((2 * D, D)), full((D,)), full((2 * D, D)), full((D,)),
            full((2 * D, D)), full((D,)), full((2 * D, D)), full((D,)),
            full((2 * D, D)), full((D,)),
            full((D, D)), full((D,)), full((D, D)), full((D,)),
            full((D, D)), full((D,)), full((D, D)), full((D,)),
            full((D,)), full((D,)),
            full((64, 128)), full((128,)), full((128, 64)), full((64,)),
        ],
        out_specs=[
            pl.BlockSpec((1, N2, D), lambda b: (b, 0, 0)),
            pl.BlockSpec((1, 1, 2 * D + 2), lambda b: (b, 0, 0)),
        ],
        out_shape=[
            jax.ShapeDtypeStruct((B, N2, D), jnp.float32),
            jax.ShapeDtypeStruct((B, 1, 2 * D + 2), jnp.float32),
        ],
    )(sc_out, hn, numerical.reshape(B, 1, -1), stage.reshape(B, 1, 2), maskcol,
      g['gate_W'], g['gate_b'], g['update_W'], g['update_b'],
      params['attn_q']['W'], params['attn_q']['b'],
      params['attn_k']['W'], params['attn_k']['b'],
      params['attn_v']['W'], params['attn_v']['b'],
      m['Wq'], m['bq'], m['Wk'], m['bk'], m['Wv'], m['bv'], m['Wo'], m['bo'],
      params['ln1']['g'], params['ln1']['b'],
      ne[0]['W'], ne[0]['b'], ne[1]['W'], ne[1]['b'])


def kernel(numerical, node_feature, edge_index_dis, edge_index_od, node_mask, stage, params):
    nf_pad = jnp.pad(node_feature, ((0, 0), (0, N2 - N), (0, 0)))
    pos_pad = jnp.pad(params['pos_enc'][:, :N, :], ((0, 0), (0, N2 - N), (0, 0)))
    zx, hn = _run_pre(nf_pad, pos_pad, params['node_enc'],
                      params['edge_fc1'][-1], params['edge_fc2'][-1])
    i0, i1, s0, s1 = _prep_edges(edge_index_dis, edge_index_od)
    zeros_in = jnp.zeros((80, W), jnp.float32)
    sc_out = _get_sc_call()(zx.reshape(2 * B * NP, W), i0, i1, s0, s1, zeros_in)
    maskcol = nf_pad[0:1, :, -5]
    h_att, sv = _run_post(sc_out[:, :, :N2, :], hn,
                          numerical.reshape(B, -1), stage, maskcol, params)
    return (h_att[:, :N, :], sv[:, 0, :], node_mask, stage)


# speedup vs baseline: 20.4564x; 1.0808x over previous
import functools

import jax
import jax.numpy as jnp
import numpy as np
from jax import lax
from jax.experimental import pallas as pl
from jax.experimental.pallas import tpu as pltpu
from jax.experimental.pallas import tpu_sc as plsc

EPS = 1e-06
N_HEADS = 8
B = 8
N = 1177
E = 40000
D = 64
NP = 1280
N2 = 1184
W = 80
N_TILES = 16
CHUNK = 128
N_CHUNKS = 20
E_TILE = CHUNK * N_CHUNKS
E_PAD = E_TILE * N_TILES
WAVE = 4
NEG = -1e30


def _pre_body(nf_ref, pos_ref, nw_ref, nb_ref,
              a1_ref, a1b_ref, b1_ref, b1b_ref,
              a2_ref, a2b_ref, b2_ref, b2b_ref,
              zx_ref, hn_ref):
    nf = nf_ref[0]
    h = jnp.tanh(nf @ nw_ref[...] + nb_ref[...]) + pos_ref[0]
    hn_ref[0] = h
    rowmask = lax.broadcasted_iota(jnp.int32, (N2, W), 0) < N
    widen = [jnp.ones((N2, 16), jnp.float32)]
    if W > D + 16:
        widen.append(jnp.zeros((N2, W - D - 16), jnp.float32))
    z1 = jnp.tanh(jnp.tanh(h @ a1_ref[...] + a1b_ref[...]) @ b1_ref[...] + b1b_ref[...])
    z2 = jnp.tanh(jnp.tanh(h @ a2_ref[...] + a2b_ref[...]) @ b2_ref[...] + b2b_ref[...])
    pad = jnp.zeros((NP - N2, W), jnp.float32)
    zx_ref[0, 0, :N2, :] = jnp.where(
        rowmask, jnp.concatenate([z1] + widen, axis=-1), 0.0)
    zx_ref[0, 0, N2:, :] = pad
    zx_ref[0, 1, :N2, :] = jnp.where(
        rowmask, jnp.concatenate([z2] + widen, axis=-1), 0.0)
    zx_ref[0, 1, N2:, :] = pad


def _run_pre(nf_pad, pos_pad, node_enc, fc1, fc2):
    full = lambda shape: pl.BlockSpec(shape, lambda b: tuple(0 for _ in shape))
    return pl.pallas_call(
        _pre_body,
        grid=(B,),
        in_specs=[
            pl.BlockSpec((1, N2, 32), lambda b: (b, 0, 0)),
            pl.BlockSpec((1, N2, D), lambda b: (0, 0, 0)),
            full((32, D)), full((D,)),
            full((D, D)), full((D,)), full((D, D)), full((D,)),
            full((D, D)), full((D,)), full((D, D)), full((D,)),
        ],
        out_specs=[
            pl.BlockSpec((1, 2, NP, W), lambda b: (b, 0, 0, 0)),
            pl.BlockSpec((1, N2, D), lambda b: (b, 0, 0)),
        ],
        out_shape=[
            jax.ShapeDtypeStruct((B, 2, NP, W), jnp.float32),
            jax.ShapeDtypeStruct((B, N2, D), jnp.float32),
        ],
    )(nf_pad, pos_pad, node_enc['W'], node_enc['b'],
      fc1[0]['W'], fc1[0]['b'], fc1[1]['W'], fc1[1]['b'],
      fc2[0]['W'], fc2[0]['b'], fc2[1]['W'], fc2[1]['b'])


def _sc_body(zx_hbm, i0_hbm, i1_hbm, s0_hbm, s1_hbm, zz_hbm, out_hbm,
             g0_v, g1_v, l0_v, l1_v, r0x, r1x, r0y, r1y, acc_sh,
             sgx0, sgx1, sgy0, sgy1, ssx0, ssx1, ssy0, ssy1):
    c = lax.axis_index("c")
    t = lax.axis_index("s")

    def wait_gather(idx_row, dst, sem):
        pltpu.make_async_copy(zx_hbm.at[idx_row], dst, sem).wait()

    def wait_scatter(dst, sem):
        pltpu.make_async_copy(zx_hbm.at[pl.ds(0, CHUNK)], dst, sem).wait()

    def do_pair(q):
        pltpu.sync_copy(i0_hbm.at[c, q, t], g0_v)
        pltpu.sync_copy(i1_hbm.at[c, q, t], g1_v)
        pltpu.sync_copy(s0_hbm.at[c, q, t], l0_v)
        pltpu.sync_copy(s1_hbm.at[c, q, t], l1_v)

        pltpu.async_copy(zx_hbm.at[g0_v.at[0]], r0x, sgx0)
        pltpu.async_copy(zx_hbm.at[g1_v.at[0]], r1x, sgx1)

        @pl.loop(0, N_CHUNKS // 2)
        def _k(k):
            ch0 = 2 * k
            ch1 = 2 * k + 1
            wait_gather(g0_v.at[ch0], r0x, sgx0)
            wait_gather(g1_v.at[ch0], r1x, sgx1)
            pltpu.async_copy(r1x, acc_sh.at[l0_v.at[ch0]], ssx0, add=True)
            pltpu.async_copy(r0x, acc_sh.at[l1_v.at[ch0]], ssx1, add=True)

            @pl.when(k > 0)
            def _():
                wait_scatter(r0y, ssy0)
                wait_scatter(r1y, ssy1)

            pltpu.async_copy(zx_hbm.at[g0_v.at[ch1]], r0y, sgy0)
            pltpu.async_copy(zx_hbm.at[g1_v.at[ch1]], r1y, sgy1)
            wait_gather(g0_v.at[ch1], r0y, sgy0)
            wait_gather(g1_v.at[ch1], r1y, sgy1)
            pltpu.async_copy(r1y, acc_sh.at[l0_v.at[ch1]], ssy0, add=True)
            pltpu.async_copy(r0y, acc_sh.at[l1_v.at[ch1]], ssy1, add=True)
            wait_scatter(r0x, ssx0)
            wait_scatter(r1x, ssx1)

            @pl.when(k < N_CHUNKS // 2 - 1)
            def _():
                pltpu.async_copy(zx_hbm.at[g0_v.at[ch0 + 2]], r0x, sgx0)
                pltpu.async_copy(zx_hbm.at[g1_v.at[ch0 + 2]], r1x, sgx1)

        wait_scatter(r0y, ssy0)
        wait_scatter(r1y, ssy1)

    for wave in range(2):
        for q in range(WAVE):
            base = q * NP + t * 80
            pltpu.sync_copy(zz_hbm, acc_sh.at[pl.ds(base, 80)])
        plsc.subcore_barrier()
        for q in range(WAVE):
            do_pair(wave * WAVE + q)
        plsc.subcore_barrier()
        for q in range(WAVE):
            base = q * NP + t * 80
            pltpu.sync_copy(acc_sh.at[pl.ds(base, 80)],
                            out_hbm.at[wave * WAVE + q, c, pl.ds(t * 80, 80)])


@functools.lru_cache(maxsize=1)
def _get_sc_call():
    return functools.partial(
        pl.kernel,
        out_type=jax.ShapeDtypeStruct((B, 2, NP, W), jnp.float32),
        mesh=plsc.VectorSubcoreMesh(core_axis_name="c", subcore_axis_name="s"),
        compiler_params=pltpu.CompilerParams(use_tc_tiling_on_sc=False),
        scratch_types=[
            pltpu.VMEM((N_CHUNKS, CHUNK), jnp.int32),
            pltpu.VMEM((N_CHUNKS, CHUNK), jnp.int32),
            pltpu.VMEM((N_CHUNKS, CHUNK), jnp.int32),
            pltpu.VMEM((N_CHUNKS, CHUNK), jnp.int32),
            pltpu.VMEM((CHUNK, W), jnp.float32),
            pltpu.VMEM((CHUNK, W), jnp.float32),
            pltpu.VMEM((CHUNK, W), jnp.float32),
            pltpu.VMEM((CHUNK, W), jnp.float32),
            pltpu.VMEM_SHARED((WAVE * NP, W), jnp.float32),
            pltpu.SemaphoreType.DMA,
            pltpu.SemaphoreType.DMA,
            pltpu.SemaphoreType.DMA,
            pltpu.SemaphoreType.DMA,
            pltpu.SemaphoreType.DMA,
            pltpu.SemaphoreType.DMA,
            pltpu.SemaphoreType.DMA,
            pltpu.SemaphoreType.DMA,
        ],
    )(_sc_body)


def _prep_edges(edge_index_dis, edge_index_od):
    def one(ei, s):
        pad = jnp.full((B, E_PAD - E, 2), N, jnp.int32)
        eip = jnp.concatenate([ei.astype(jnp.int32), pad], axis=1)
        gbase = ((2 * jnp.arange(B, dtype=jnp.int32) + s) * NP)[:, None]
        lbase = ((jnp.arange(B, dtype=jnp.int32) % WAVE) * NP)[:, None]
        shp = (B, N_TILES, N_CHUNKS, CHUNK)
        return ((eip[:, :, 0] + gbase).reshape(shp),
                (eip[:, :, 1] + gbase).reshape(shp),
                (eip[:, :, 0] + lbase).reshape(shp),
                (eip[:, :, 1] + lbase).reshape(shp))
    d = one(edge_index_dis, 0)
    o = one(edge_index_od, 1)
    return tuple(jnp.stack([d[k], o[k]], axis=0) for k in range(4))


def _post_body(sc_ref, hn_ref, num_ref, stage_ref, mc_ref,
               gw_ref, gb_ref, uw_ref, ub_ref,
               qw_ref, qb_ref, kw_ref, kb_ref, vw_ref, vb_ref,
               mqw_ref, mqb_ref, mkw_ref, mkb_ref, mvw_ref, mvb_ref,
               mow_ref, mob_ref, lg_ref, lb_ref,
               n1w_ref, n1b_ref, n2w_ref, n2b_ref,
               hatt_ref, sv_ref):
    hnum = jnp.tanh(num_ref[0] @ n1w_ref[...] + n1b_ref[...])
    hnum = jnp.tanh(hnum @ n2w_ref[...] + n2b_ref[...])
    cur = hn_ref[0]

    hs = []
    for s in range(2):
        acc = sc_ref[0, s]
        comb = acc[:, :D] / (acc[:, D:D + 1] + EPS)
        cat = jnp.concatenate([cur, comb], axis=-1)
        gate = jnp.tanh(cat @ gw_ref[...] + gb_ref[...])
        upd = jnp.tanh(cat @ uw_ref[...] + ub_ref[...])
        hs.append(gate * upd + (1.0 - gate) * cur)
    hcat = jnp.concatenate(hs, axis=-1)

    q = hcat @ qw_ref[...] + qb_ref[...]
    k = hcat @ kw_ref[...] + kb_ref[...]
    v = hcat @ vw_ref[...] + vb_ref[...]
    qp = q @ mqw_ref[...] + mqb_ref[...]
    kp = k @ mkw_ref[...] + mkb_ref[...]
    vp = v @ mvw_ref[...] + mvb_ref[...]

    keymaskT = (mc_ref[...] > 0.0)
    dh = D // N_HEADS
    scale = 1.0 / float(np.sqrt(dh))
    outsT = []
    for h in range(N_HEADS):
        sl = slice(h * dh, (h + 1) * dh)
        sT = lax.dot_general(kp[:, sl], qp[:, sl],
                             (((1,), (1,)), ((), ()))) * scale
        sT = jnp.where(keymaskT, sT, NEG)
        m = jnp.max(sT, axis=0, keepdims=True)
        e = jnp.exp(sT - m)
        a = e / jnp.sum(e, axis=0, keepdims=True)
        outsT.append(lax.dot_general(vp[:, sl], a,
                                     (((0,), (0,)), ((), ()))))
    att = lax.dot_general(jnp.concatenate(outsT, axis=0), mow_ref[...],
                          (((0,), (0,)), ((), ()))) + mob_ref[...]

    mu = att.mean(axis=-1, keepdims=True)
    var = ((att - mu) ** 2).mean(axis=-1, keepdims=True)
    h_att = (att - mu) / jnp.sqrt(var + 1e-05) * lg_ref[...] + lb_ref[...]
    hatt_ref[0] = h_att

    rmask = lax.broadcasted_iota(jnp.int32, (N2, D), 0) < N
    hmean = jnp.sum(jnp.where(rmask, h_att, 0.0), axis=0, keepdims=True) / float(N)
    sv_ref[0, 0, :D] = hnum[0]
    sv_ref[0, 0, D:2 * D] = hmean[0]
    sv_ref[0, 0, 2 * D:] = stage_ref[0, 0]


def _run_post(sc_out, hn, numerical, stage, maskcol, params):
    full = lambda shape: pl.BlockSpec(shape, lambda b: tuple(0 for _ in shape))
    g = params['gated']
    m = params['mha1']
    ne = params['num_enc']
    return pl.pallas_call(
        _post_body,
        grid=(B,),
        in_specs=[
            pl.BlockSpec((1, 2, N2, W), lambda b: (b, 0, 0, 0)),
            pl.BlockSpec((1, N2, D), lambda b: (b, 0, 0)),
            pl.BlockSpec((1, 1, 64), lambda b: (b, 0, 0)),
            pl.BlockSpec((1, 1, 2), lambda b: (b, 0, 0)),
            full((N2, 1)),
            full((2 * D, D)), full((D,)), full((2 * D, D)), full((D,)),
            full((2 * D, D)), full((D,)), full((2 * D, D)), full((D,)),
            full((2 * D, D)), full((D,)),
            full((D, D)), full((D,)), full((D, D)), full((D,)),
            full((D, D)), full((D,)), full((D, D)), full((D,)),
            full((D,)), full((D,)),
            full((64, 128)), full((128,)), full((128, 64)), full((64,)),
        ],
        out_specs=[
            pl.BlockSpec((1, N2, D), lambda b: (b, 0, 0)),
            pl.BlockSpec((1, 1, 2 * D + 2), lambda b: (b, 0, 0)),
        ],
        out_shape=[
            jax.ShapeDtypeStruct((B, N2, D), jnp.float32),
            jax.ShapeDtypeStruct((B, 1, 2 * D + 2), jnp.float32),
        ],
    )(sc_out, hn, numerical.reshape(B, 1, -1), stage.reshape(B, 1, 2), maskcol,
      g['gate_W'], g['gate_b'], g['update_W'], g['update_b'],
      params['attn_q']['W'], params['attn_q']['b'],
      params['attn_k']['W'], params['attn_k']['b'],
      params['attn_v']['W'], params['attn_v']['b'],
      m['Wq'], m['bq'], m['Wk'], m['bk'], m['Wv'], m['bv'], m['Wo'], m['bo'],
      params['ln1']['g'], params['ln1']['b'],
      ne[0]['W'], ne[0]['b'], ne[1]['W'], ne[1]['b'])


def kernel(numerical, node_feature, edge_index_dis, edge_index_od, node_mask, stage, params):
    nf_pad = jnp.pad(node_feature, ((0, 0), (0, N2 - N), (0, 0)))
    pos_pad = jnp.pad(params['pos_enc'][:, :N, :], ((0, 0), (0, N2 - N), (0, 0)))
    zx, hn = _run_pre(nf_pad, pos_pad, params['node_enc'],
                      params['edge_fc1'][-1], params['edge_fc2'][-1])
    i0, i1, s0, s1 = _prep_edges(edge_index_dis, edge_index_od)
    zeros_in = jnp.zeros((80, W), jnp.float32)
    sc_out = _get_sc_call()(zx.reshape(2 * B * NP, W), i0, i1, s0, s1, zeros_in)
    maskcol = nf_pad[0, :, -5].reshape(N2, 1)
    h_att, sv = _run_post(sc_out[:, :, :N2, :], hn,
                          numerical.reshape(B, -1), stage, maskcol, params)
    return (h_att[:, :N, :], sv[:, 0, :], node_mask, stage)
